# baseline (speedup 1.0000x reference)
import functools

import jax
import jax.numpy as jnp
from jax import lax
from jax.experimental import pallas as pl
from jax.experimental.pallas import tpu as pltpu
from jax.experimental.pallas import tpu_sc as plsc

D = 64
SCALE = 8.0

NC = 2
NS = 16
NW = NC * NS

CHUNK = 512
IDX_ROWS = CHUNK // 128


def _make_kernel(B):
    b_per_w = B // NW
    n_chunks = b_per_w // CHUNK
    mesh = plsc.VectorSubcoreMesh(core_axis_name="c", subcore_axis_name="s")

    @functools.partial(
        pl.kernel,
        mesh=mesh,
        out_type=jax.ShapeDtypeStruct((B, D), jnp.float32),
        compiler_params=pltpu.CompilerParams(use_tc_tiling_on_sc=False),
        scratch_types=[
            pltpu.VMEM((IDX_ROWS, 128), jnp.int32),
            pltpu.VMEM((CHUNK, D), jnp.float32),
            pltpu.SemaphoreType.DMA,
        ],
    )
    def emb_kernel(x_hbm, table_hbm, out_hbm, idx_v, rows_v, sem):
        wid = lax.axis_index("s") * NC + lax.axis_index("c")
        row_base = wid * (b_per_w // 128)

        def chunk_body(ci, carry):
            row_off = row_base + ci * IDX_ROWS
            pltpu.sync_copy(x_hbm.at[pl.ds(row_off, IDX_ROWS)], idx_v)
            cps = [
                pltpu.async_copy(
                    table_hbm.at[idx_v.at[j]],
                    rows_v.at[pl.ds(j * 128, 128)],
                    sem,
                )
                for j in range(IDX_ROWS)
            ]
            for cp in cps:
                cp.wait()

            def mul_body(i, c2):
                for j in range(D // 16):
                    v = rows_v[i, pl.ds(j * 16, 16)]
                    rows_v[i, pl.ds(j * 16, 16)] = v * SCALE
                return c2

            lax.fori_loop(0, CHUNK, mul_body, 0)

            out_off = wid * b_per_w + ci * CHUNK
            pltpu.sync_copy(rows_v, out_hbm.at[pl.ds(out_off, CHUNK)])
            return carry

        lax.fori_loop(0, n_chunks, chunk_body, 0)

    return emb_kernel


def kernel(x, table):
    rows, cols = x.shape
    B = rows * cols
    x2d = x.reshape(B // 128, 128).astype(jnp.int32)
    out = _make_kernel(B)(x2d, table)
    return out.reshape(rows, cols, D)


# speedup vs baseline: 1.4088x; 1.4088x over previous
import functools

import jax
import jax.numpy as jnp
from jax import lax
from jax.experimental import pallas as pl
from jax.experimental.pallas import tpu as pltpu
from jax.experimental.pallas import tpu_sc as plsc

D = 64
SCALE = 8.0

NC = 2
NS = 16
NW = NC * NS

CHUNK = 512
IDX_ROWS = CHUNK // 128
NBUF = 2


def _make_kernel(B):
    b_per_w = B // NW
    n_chunks = b_per_w // CHUNK
    assert n_chunks % NBUF == 0
    mesh = plsc.VectorSubcoreMesh(core_axis_name="c", subcore_axis_name="s")

    @functools.partial(
        pl.kernel,
        mesh=mesh,
        out_type=jax.ShapeDtypeStruct((B, 2 * D), jnp.float32),
        compiler_params=pltpu.CompilerParams(use_tc_tiling_on_sc=False),
        scratch_types=[
            pltpu.VMEM((NBUF, IDX_ROWS, 128), jnp.int32),
            pltpu.VMEM((NBUF, CHUNK, D), jnp.float32),
            pltpu.SemaphoreType.DMA,
            pltpu.SemaphoreType.DMA,
        ],
    )
    def emb_kernel(x_hbm, table_hbm, out_hbm, idx_v, rows_v, gsem, osem):
        wid = lax.axis_index("s") * NC + lax.axis_index("c")
        row_base = wid * (b_per_w // 128)
        out_base = wid * b_per_w

        def fire(ci, slot):
            pltpu.sync_copy(
                x_hbm.at[pl.ds(row_base + ci * IDX_ROWS, IDX_ROWS)],
                idx_v.at[slot],
            )
            for j in range(IDX_ROWS):
                pltpu.async_copy(
                    table_hbm.at[idx_v.at[slot, j]],
                    rows_v.at[slot, pl.ds(j * 128, 128)],
                    gsem,
                )

        def drain_gathers(slot):
            for j in range(IDX_ROWS):
                pltpu.make_async_copy(
                    table_hbm.at[idx_v.at[slot, j]],
                    rows_v.at[slot, pl.ds(j * 128, 128)],
                    gsem,
                ).wait()

        def scale_slot(slot):
            def mul_body(i, c2):
                for j in range(D // 16):
                    v = rows_v[slot, i, pl.ds(j * 16, 16)]
                    rows_v[slot, i, pl.ds(j * 16, 16)] = v * SCALE
                return c2

            lax.fori_loop(0, CHUNK, mul_body, 0)

        def write_slot(ci, slot):
            pltpu.async_copy(
                rows_v.at[slot],
                out_hbm.at[pl.ds(out_base + ci * CHUNK, CHUNK), pl.ds(0, D)],
                osem,
            )

        def drain_write(ci, slot):
            pltpu.make_async_copy(
                rows_v.at[slot],
                out_hbm.at[pl.ds(out_base + ci * CHUNK, CHUNK), pl.ds(0, D)],
                osem,
            ).wait()

        fire(0, 0)

        def chunk_body(ci, carry):
            slot = lax.rem(ci, NBUF)
            nslot = lax.rem(ci + 1, NBUF)

            @pl.when(ci + 1 < n_chunks)
            def _():
                @pl.when(ci + 1 >= NBUF)
                def _():
                    drain_write(ci + 1 - NBUF, nslot)

                fire(ci + 1, nslot)

            drain_gathers(slot)
            scale_slot(slot)
            write_slot(ci, slot)
            return carry

        lax.fori_loop(0, n_chunks, chunk_body, 0, unroll=NBUF)
        drain_write(n_chunks - 2, (n_chunks - 2) % NBUF)
        drain_write(n_chunks - 1, (n_chunks - 1) % NBUF)

    return emb_kernel


def kernel(x, table):
    rows, cols = x.shape
    B = rows * cols
    x2d = x.reshape(B // 128, 128).astype(jnp.int32)
    out = _make_kernel(B)(x2d, table)
    return out[:, :D].reshape(rows, cols, D)


# speedup vs baseline: 1.4099x; 1.0008x over previous
import functools

import jax
import jax.numpy as jnp
from jax import lax
from jax.experimental import pallas as pl
from jax.experimental.pallas import tpu as pltpu
from jax.experimental.pallas import tpu_sc as plsc

D = 64
SCALE = 8.0

NC = 2
NS = 16
NW = NC * NS

CHUNK = 512
IDX_ROWS = CHUNK // 128
NBUF = 3


def _make_kernel(B):
    b_per_w = B // NW
    n_chunks = b_per_w // CHUNK
    mesh = plsc.VectorSubcoreMesh(core_axis_name="c", subcore_axis_name="s")

    @functools.partial(
        pl.kernel,
        mesh=mesh,
        out_type=jax.ShapeDtypeStruct((B, 2 * D), jnp.float32),
        compiler_params=pltpu.CompilerParams(use_tc_tiling_on_sc=False),
        scratch_types=[
            pltpu.VMEM((NBUF, IDX_ROWS, 128), jnp.int32),
            pltpu.VMEM((NBUF, CHUNK, D), jnp.float32),
            pltpu.SemaphoreType.DMA,
            pltpu.SemaphoreType.DMA,
        ],
    )
    def emb_kernel(x_hbm, table_hbm, out_hbm, idx_v, rows_v, gsem, osem):
        wid = lax.axis_index("s") * NC + lax.axis_index("c")
        row_base = wid * (b_per_w // 128)
        out_base = wid * b_per_w

        def fire(ci, slot):
            pltpu.sync_copy(
                x_hbm.at[pl.ds(row_base + ci * IDX_ROWS, IDX_ROWS)],
                idx_v.at[slot],
            )
            for j in range(IDX_ROWS):
                pltpu.async_copy(
                    table_hbm.at[idx_v.at[slot, j]],
                    rows_v.at[slot, pl.ds(j * 128, 128)],
                    gsem,
                )

        def drain_gathers(slot):
            for j in range(IDX_ROWS):
                pltpu.make_async_copy(
                    table_hbm.at[idx_v.at[slot, j]],
                    rows_v.at[slot, pl.ds(j * 128, 128)],
                    gsem,
                ).wait()

        def scale_slot(slot):
            def mul_body(i, c2):
                for j in range(D // 16):
                    v = rows_v[slot, i, pl.ds(j * 16, 16)]
                    rows_v[slot, i, pl.ds(j * 16, 16)] = v * SCALE
                return c2

            lax.fori_loop(0, CHUNK, mul_body, 0)

        def write_slot(ci, slot):
            pltpu.async_copy(
                rows_v.at[slot],
                out_hbm.at[pl.ds(out_base + ci * CHUNK, CHUNK), pl.ds(0, D)],
                osem,
            )

        def drain_write(ci, slot):
            pltpu.make_async_copy(
                rows_v.at[slot],
                out_hbm.at[pl.ds(out_base + ci * CHUNK, CHUNK), pl.ds(0, D)],
                osem,
            ).wait()

        for p in range(NBUF - 1):
            fire(p, p)

        def chunk_body(ci, carry):
            slot = lax.rem(ci, NBUF)
            nslot = lax.rem(ci + NBUF - 1, NBUF)

            @pl.when(ci + NBUF - 1 < n_chunks)
            def _():
                @pl.when(ci >= 1)
                def _():
                    drain_write(ci - 1, nslot)

                fire(ci + NBUF - 1, nslot)

            drain_gathers(slot)
            scale_slot(slot)
            write_slot(ci, slot)
            return carry

        lax.fori_loop(0, n_chunks, chunk_body, 0, unroll=NBUF)
        for back in range(NBUF, 0, -1):
            drain_write(n_chunks - back, (n_chunks - back) % NBUF)

    return emb_kernel


def kernel(x, table):
    rows, cols = x.shape
    B = rows * cols
    x2d = x.reshape(B // 128, 128).astype(jnp.int32)
    out = _make_kernel(B)(x2d, table)
    return out[:, :D].reshape(rows, cols, D)


# speedup vs baseline: 1.4533x; 1.0308x over previous
import functools

import jax
import jax.numpy as jnp
from jax import lax
from jax.experimental import pallas as pl
from jax.experimental.pallas import tpu as pltpu
from jax.experimental.pallas import tpu_sc as plsc

D = 64
SCALE = 8.0

NC = 2
NS = 16
NW = NC * NS

CHUNK = 512
IDX_ROWS = CHUNK // 128
NBUF = 2


def _make_kernel(B):
    b_per_w = B // NW
    idx_rows_w = b_per_w // 128
    n_chunks = b_per_w // CHUNK
    mesh = plsc.VectorSubcoreMesh(core_axis_name="c", subcore_axis_name="s")

    @functools.partial(
        pl.kernel,
        mesh=mesh,
        out_type=jax.ShapeDtypeStruct((B, 2 * D), jnp.float32),
        compiler_params=pltpu.CompilerParams(use_tc_tiling_on_sc=False),
        scratch_types=[
            pltpu.VMEM((idx_rows_w, 128), jnp.int32),
            pltpu.VMEM((NBUF, CHUNK, D), jnp.float32),
            pltpu.SemaphoreType.DMA,
            pltpu.SemaphoreType.DMA,
        ],
    )
    def emb_kernel(x_hbm, table_hbm, out_hbm, idx_v, rows_v, gsem, osem):
        wid = lax.axis_index("s") * NC + lax.axis_index("c")
        out_base = wid * b_per_w

        pltpu.sync_copy(x_hbm.at[pl.ds(wid * idx_rows_w, idx_rows_w)], idx_v)

        def fire(ci, slot):
            for j in range(IDX_ROWS):
                pltpu.async_copy(
                    table_hbm.at[idx_v.at[ci * IDX_ROWS + j]],
                    rows_v.at[slot, pl.ds(j * 128, 128)],
                    gsem,
                )

        def drain_gathers(ci, slot):
            for j in range(IDX_ROWS):
                pltpu.make_async_copy(
                    table_hbm.at[idx_v.at[ci * IDX_ROWS + j]],
                    rows_v.at[slot, pl.ds(j * 128, 128)],
                    gsem,
                ).wait()

        def scale_slot(slot):
            def mul_body(i, c2):
                for j in range(D // 16):
                    v = rows_v[slot, i, pl.ds(j * 16, 16)]
                    rows_v[slot, i, pl.ds(j * 16, 16)] = v * SCALE
                return c2

            lax.fori_loop(0, CHUNK, mul_body, 0)

        def write_slot(ci, slot):
            pltpu.async_copy(
                rows_v.at[slot],
                out_hbm.at[pl.ds(out_base + ci * CHUNK, CHUNK), pl.ds(0, D)],
                osem,
            )

        def drain_write(ci, slot):
            pltpu.make_async_copy(
                rows_v.at[slot],
                out_hbm.at[pl.ds(out_base + ci * CHUNK, CHUNK), pl.ds(0, D)],
                osem,
            ).wait()

        for p in range(NBUF - 1):
            fire(p, p)

        def chunk_body(ci, carry):
            slot = lax.rem(ci, NBUF)
            nslot = lax.rem(ci + NBUF - 1, NBUF)

            @pl.when(ci + NBUF - 1 < n_chunks)
            def _():
                @pl.when(ci >= 1)
                def _():
                    drain_write(ci - 1, nslot)

                fire(ci + NBUF - 1, nslot)

            drain_gathers(ci, slot)
            scale_slot(slot)
            write_slot(ci, slot)
            return carry

        lax.fori_loop(0, n_chunks, chunk_body, 0, unroll=NBUF)
        for back in range(NBUF, 0, -1):
            drain_write(n_chunks - back, (n_chunks - back) % NBUF)

    return emb_kernel


def kernel(x, table):
    rows, cols = x.shape
    B = rows * cols
    x2d = x.reshape(B // 128, 128).astype(jnp.int32)
    out = _make_kernel(B)(x2d, table)
    return out[:, :D].reshape(rows, cols, D)


# speedup vs baseline: 1.5100x; 1.0390x over previous
import functools

import jax
import jax.numpy as jnp
from jax import lax
from jax.experimental import pallas as pl
from jax.experimental.pallas import tpu as pltpu
from jax.experimental.pallas import tpu_sc as plsc

D = 64
SCALE = 8.0

NC = 2
NS = 16
NW = NC * NS

CHUNK = 512
IDX_ROWS = CHUNK // 128
NBUF = 3


def _make_kernel(B):
    b_per_w = B // NW
    idx_rows_w = b_per_w // 128
    n_chunks = b_per_w // CHUNK
    mesh = plsc.VectorSubcoreMesh(core_axis_name="c", subcore_axis_name="s")

    @functools.partial(
        pl.kernel,
        mesh=mesh,
        out_type=jax.ShapeDtypeStruct((B, 2 * D), jnp.float32),
        compiler_params=pltpu.CompilerParams(use_tc_tiling_on_sc=False),
        scratch_types=[
            pltpu.VMEM((idx_rows_w, 128), jnp.int32),
            pltpu.VMEM((NBUF, CHUNK, D), jnp.float32),
            pltpu.SemaphoreType.DMA,
            pltpu.SemaphoreType.DMA,
        ],
    )
    def emb_kernel(x_hbm, table_hbm, out_hbm, idx_v, rows_v, gsem, osem):
        wid = lax.axis_index("s") * NC + lax.axis_index("c")
        out_base = wid * b_per_w

        pltpu.sync_copy(x_hbm.at[pl.ds(wid * idx_rows_w, idx_rows_w)], idx_v)

        def fire(ci, slot):
            for j in range(IDX_ROWS):
                pltpu.async_copy(
                    table_hbm.at[idx_v.at[ci * IDX_ROWS + j]],
                    rows_v.at[slot, pl.ds(j * 128, 128)],
                    gsem,
                )

        def drain_gathers(ci, slot):
            for j in range(IDX_ROWS):
                pltpu.make_async_copy(
                    table_hbm.at[idx_v.at[ci * IDX_ROWS + j]],
                    rows_v.at[slot, pl.ds(j * 128, 128)],
                    gsem,
                ).wait()

        def scale_slot(slot):
            def mul_body(i, c2):
                for r in range(2):
                    for j in range(D // 16):
                        v = rows_v[slot, 2 * i + r, pl.ds(j * 16, 16)]
                        rows_v[slot, 2 * i + r, pl.ds(j * 16, 16)] = v * SCALE
                return c2

            lax.fori_loop(0, CHUNK // 2, mul_body, 0)

        def write_slot(ci, slot):
            pltpu.async_copy(
                rows_v.at[slot],
                out_hbm.at[pl.ds(out_base + ci * CHUNK, CHUNK), pl.ds(0, D)],
                osem,
            )

        def drain_write(ci, slot):
            pltpu.make_async_copy(
                rows_v.at[slot],
                out_hbm.at[pl.ds(out_base + ci * CHUNK, CHUNK), pl.ds(0, D)],
                osem,
            ).wait()

        for p in range(NBUF - 1):
            fire(p, p)

        def chunk_body(ci, carry):
            slot = lax.rem(ci, NBUF)
            nslot = lax.rem(ci + NBUF - 1, NBUF)

            @pl.when(ci + NBUF - 1 < n_chunks)
            def _():
                @pl.when(ci >= 1)
                def _():
                    drain_write(ci - 1, nslot)

                fire(ci + NBUF - 1, nslot)

            drain_gathers(ci, slot)
            scale_slot(slot)
            write_slot(ci, slot)
            return carry

        lax.fori_loop(0, n_chunks, chunk_body, 0, unroll=NBUF)
        for back in range(NBUF, 0, -1):
            drain_write(n_chunks - back, (n_chunks - back) % NBUF)

    return emb_kernel


def kernel(x, table):
    rows, cols = x.shape
    B = rows * cols
    x2d = x.reshape(B // 128, 128).astype(jnp.int32)
    out = _make_kernel(B)(x2d, table)
    return out[:, :D].reshape(rows, cols, D)


# speedup vs baseline: 1.5138x; 1.0026x over previous
import functools

import jax
import jax.numpy as jnp
from jax import lax
from jax.experimental import pallas as pl
from jax.experimental.pallas import tpu as pltpu
from jax.experimental.pallas import tpu_sc as plsc

D = 64
SCALE = 8.0

NC = 2
NS = 16
NW = NC * NS

CHUNK = 256
IDX_ROWS = CHUNK // 128
NBUF = 6


def _make_kernel(B):
    b_per_w = B // NW
    idx_rows_w = b_per_w // 128
    n_chunks = b_per_w // CHUNK
    mesh = plsc.VectorSubcoreMesh(core_axis_name="c", subcore_axis_name="s")

    @functools.partial(
        pl.kernel,
        mesh=mesh,
        out_type=jax.ShapeDtypeStruct((B, 2 * D), jnp.float32),
        compiler_params=pltpu.CompilerParams(use_tc_tiling_on_sc=False),
        scratch_types=[
            pltpu.VMEM((idx_rows_w, 128), jnp.int32),
            pltpu.VMEM((NBUF, CHUNK, D), jnp.float32),
            pltpu.SemaphoreType.DMA,
            pltpu.SemaphoreType.DMA,
        ],
    )
    def emb_kernel(x_hbm, table_hbm, out_hbm, idx_v, rows_v, gsem, osem):
        wid = lax.axis_index("s") * NC + lax.axis_index("c")
        out_base = wid * b_per_w

        pltpu.sync_copy(x_hbm.at[pl.ds(wid * idx_rows_w, idx_rows_w)], idx_v)

        def fire(ci, slot):
            for j in range(IDX_ROWS):
                pltpu.async_copy(
                    table_hbm.at[idx_v.at[ci * IDX_ROWS + j]],
                    rows_v.at[slot, pl.ds(j * 128, 128)],
                    gsem,
                )

        def drain_gathers(ci, slot):
            for j in range(IDX_ROWS):
                pltpu.make_async_copy(
                    table_hbm.at[idx_v.at[ci * IDX_ROWS + j]],
                    rows_v.at[slot, pl.ds(j * 128, 128)],
                    gsem,
                ).wait()

        def scale_slot(slot):
            def mul_body(i, c2):
                for r in range(2):
                    for j in range(D // 16):
                        v = rows_v[slot, 2 * i + r, pl.ds(j * 16, 16)]
                        rows_v[slot, 2 * i + r, pl.ds(j * 16, 16)] = v * SCALE
                return c2

            lax.fori_loop(0, CHUNK // 2, mul_body, 0)

        def write_slot(ci, slot):
            pltpu.async_copy(
                rows_v.at[slot],
                out_hbm.at[pl.ds(out_base + ci * CHUNK, CHUNK), pl.ds(0, D)],
                osem,
            )

        def drain_write(ci, slot):
            pltpu.make_async_copy(
                rows_v.at[slot],
                out_hbm.at[pl.ds(out_base + ci * CHUNK, CHUNK), pl.ds(0, D)],
                osem,
            ).wait()

        for p in range(NBUF - 1):
            fire(p, p)

        def chunk_body(ci, carry):
            slot = lax.rem(ci, NBUF)
            nslot = lax.rem(ci + NBUF - 1, NBUF)

            @pl.when(ci + NBUF - 1 < n_chunks)
            def _():
                @pl.when(ci >= 1)
                def _():
                    drain_write(ci - 1, nslot)

                fire(ci + NBUF - 1, nslot)

            drain_gathers(ci, slot)
            scale_slot(slot)
            write_slot(ci, slot)
            return carry

        lax.fori_loop(0, n_chunks, chunk_body, 0, unroll=NBUF)
        for back in range(NBUF, 0, -1):
            drain_write(n_chunks - back, (n_chunks - back) % NBUF)

    return emb_kernel


def kernel(x, table):
    rows, cols = x.shape
    B = rows * cols
    x2d = x.reshape(B // 128, 128).astype(jnp.int32)
    out = _make_kernel(B)(x2d, table)
    return out[:, :D].reshape(rows, cols, D)
